# baseline (speedup 1.0000x reference)
import jax
import jax.numpy as jnp
from jax.experimental import pallas as pl
from jax.experimental.pallas import tpu as pltpu

RB = 256


def _gcn_block(x_ref, w_ref, b_ref, a_ref, o_ref, a_out_ref, xw_ref):
    @pl.when(pl.program_id(1) == 0)
    def _():
        xw_ref[...] = jnp.dot(
            x_ref[0], w_ref[...], preferred_element_type=jnp.float32
        )

    a_blk = a_ref[0]
    acc = jnp.dot(a_blk, xw_ref[...], preferred_element_type=jnp.float32)
    o_ref[0] = jnp.maximum(acc + b_ref[...], 0.0)
    a_out_ref[0] = a_blk


def kernel(X, A, weight, bias):
    B, N, D = X.shape
    bias2d = bias.reshape(1, D)
    grid = (B, N // RB)
    out = pl.pallas_call(
        _gcn_block,
        grid=grid,
        in_specs=[
            pl.BlockSpec((1, N, D), lambda b, j: (b, 0, 0)),
            pl.BlockSpec((D, D), lambda b, j: (0, 0)),
            pl.BlockSpec((1, D), lambda b, j: (0, 0)),
            pl.BlockSpec((1, RB, N), lambda b, j: (b, j, 0)),
        ],
        out_specs=[
            pl.BlockSpec((1, RB, D), lambda b, j: (b, j, 0)),
            pl.BlockSpec((1, RB, N), lambda b, j: (b, j, 0)),
        ],
        out_shape=[
            jax.ShapeDtypeStruct((B, N, D), jnp.float32),
            jax.ShapeDtypeStruct((B, N, N), jnp.float32),
        ],
        scratch_shapes=[pltpu.VMEM((N, D), jnp.float32)],
        compiler_params=pltpu.CompilerParams(
            dimension_semantics=("parallel", "arbitrary"),
        ),
    )(X, weight, bias2d, A)
    out, a_out = out
    return (out, a_out)


# speedup vs baseline: 1.1805x; 1.1805x over previous
import jax
import jax.numpy as jnp
from jax.experimental import pallas as pl
from jax.experimental.pallas import tpu as pltpu

RB = 1024


def _gcn_block(x_ref, w_ref, b_ref, a_ref, o_ref, a_out_ref, xw_ref):
    @pl.when(pl.program_id(1) == 0)
    def _():
        xw_ref[...] = jnp.dot(
            x_ref[0], w_ref[...], preferred_element_type=jnp.float32
        )

    a_blk = a_ref[0]
    acc = jnp.dot(a_blk, xw_ref[...], preferred_element_type=jnp.float32)
    o_ref[0] = jnp.maximum(acc + b_ref[...], 0.0)
    a_out_ref[0] = a_blk


def kernel(X, A, weight, bias):
    B, N, D = X.shape
    bias2d = bias.reshape(1, D)
    grid = (B, N // RB)
    out = pl.pallas_call(
        _gcn_block,
        grid=grid,
        in_specs=[
            pl.BlockSpec((1, N, D), lambda b, j: (b, 0, 0)),
            pl.BlockSpec((D, D), lambda b, j: (0, 0)),
            pl.BlockSpec((1, D), lambda b, j: (0, 0)),
            pl.BlockSpec((1, RB, N), lambda b, j: (b, j, 0)),
        ],
        out_specs=[
            pl.BlockSpec((1, RB, D), lambda b, j: (b, j, 0)),
            pl.BlockSpec((1, RB, N), lambda b, j: (b, j, 0)),
        ],
        out_shape=[
            jax.ShapeDtypeStruct((B, N, D), jnp.float32),
            jax.ShapeDtypeStruct((B, N, N), jnp.float32),
        ],
        scratch_shapes=[pltpu.VMEM((N, D), jnp.float32)],
        compiler_params=pltpu.CompilerParams(
            dimension_semantics=("parallel", "arbitrary"),
        ),
    )(X, weight, bias2d, A)
    out, a_out = out
    return (out, a_out)
